# baseline (speedup 1.0000x reference)
import functools

import jax
import jax.numpy as jnp
from jax import lax
from jax.experimental import pallas as pl
from jax.experimental.pallas import tpu as pltpu
from jax.experimental.pallas import tpu_sc as plsc

N_SUB = 16
N_CORE = 2


def _tc1_body(x_ref, wl_ref, wr_ref, o1_ref, o2_ref):
    xb = x_ref[...]
    y1 = lax.dot_general(xb, wl_ref[...], (((1,), (1,)), ((), ())),
                         preferred_element_type=jnp.float32)
    yr = lax.dot_general(xb, wr_ref[...], (((1,), (1,)), ((), ())),
                         preferred_element_type=jnp.float32)
    o1_ref[...] = y1
    o2_ref[...] = yr


def _tc2_body(p_ref, c_ref, yr_ref, b_ref, h_ref, inv_ref):
    p = p_ref[...]
    sm = p[0] + p[1]
    cnt = c_ref[0] + c_ref[1]
    inv = 1.0 / jnp.maximum(cnt, 1.0)
    h_ref[...] = jnp.maximum(sm * inv + b_ref[...] + yr_ref[...], 0.0)
    inv_ref[...] = inv[:, :1]


def _tc3_body(q_ref, inv_ref, h_ref, wl_ref, wr_ref, b_ref, o_ref):
    q = q_ref[...]
    m2 = (q[0] + q[1]) * inv_ref[...]
    h = h_ref[...]
    out = lax.dot_general(m2, wl_ref[...], (((1,), (1,)), ((), ())),
                          preferred_element_type=jnp.float32)
    out += lax.dot_general(h, wr_ref[...], (((1,), (1,)), ((), ())),
                           preferred_element_type=jnp.float32)
    o_ref[...] = out + b_ref[...]


def _make_sc_segsum(n_nodes, d, n_chunks, chunk, with_counts):
    rows_per_sub = n_nodes // N_SUB
    mesh = plsc.VectorSubcoreMesh(core_axis_name="c", subcore_axis_name="s")

    out_type = [jax.ShapeDtypeStruct((N_CORE * n_nodes, d), jnp.float32)]
    scratch = [
        pltpu.VMEM((n_chunks, chunk), jnp.int32),
        pltpu.VMEM((n_chunks, chunk), jnp.int32),
        pltpu.VMEM((chunk, d), jnp.float32),
        pltpu.VMEM((chunk, d), jnp.float32),
        pltpu.VMEM((rows_per_sub, d), jnp.float32),
        pltpu.VMEM_SHARED((n_nodes, d), jnp.float32),
        pltpu.SemaphoreType.DMA,
        pltpu.SemaphoreType.DMA,
    ]
    if with_counts:
        out_type.append(jax.ShapeDtypeStruct((N_CORE * n_nodes, 16), jnp.float32))
        scratch += [
            pltpu.VMEM((chunk, 16), jnp.float32),
            pltpu.VMEM((rows_per_sub, 16), jnp.float32),
            pltpu.VMEM_SHARED((n_nodes, 16), jnp.float32),
        ]

    @functools.partial(
        pl.kernel, mesh=mesh,
        compiler_params=pltpu.CompilerParams(use_tc_tiling_on_sc=False),
        out_type=out_type, scratch_types=scratch,
    )
    def segsum(*refs):
        if with_counts:
            (table_hbm, src_hbm, dst_hbm, out_hbm, cnt_hbm,
             src_v, dst_v, rows0, rows1, stage_v, acc, sem0, sem1,
             ones_v, cstage_v, cacc) = refs
        else:
            (table_hbm, src_hbm, dst_hbm, out_hbm,
             src_v, dst_v, rows0, rows1, stage_v, acc, sem0, sem1) = refs
        c = lax.axis_index("c")
        s = lax.axis_index("s")
        wid = c * N_SUB + s
        row0 = s * rows_per_sub

        def zbody(i, carry):
            for k in range(d // 16):
                stage_v[i, pl.ds(k * 16, 16)] = jnp.zeros((16,), jnp.float32)
            return carry
        lax.fori_loop(0, rows_per_sub, zbody, 0)
        pltpu.sync_copy(stage_v, acc.at[pl.ds(row0, rows_per_sub)])
        if with_counts:
            def czbody(i, carry):
                cstage_v[i, pl.ds(0, 16)] = jnp.zeros((16,), jnp.float32)
                return carry
            lax.fori_loop(0, rows_per_sub, czbody, 0)
            pltpu.sync_copy(cstage_v, cacc.at[pl.ds(row0, rows_per_sub)])

            def obody(i, carry):
                ones_v[i, pl.ds(0, 16)] = jnp.ones((16,), jnp.float32)
                return carry
            lax.fori_loop(0, chunk, obody, 0)

        pltpu.sync_copy(src_hbm.at[wid], src_v)
        pltpu.sync_copy(dst_hbm.at[wid], dst_v)

        plsc.subcore_barrier()

        dummy = table_hbm.at[pl.ds(0, chunk)]
        pltpu.async_copy(table_hbm.at[src_v.at[0]], rows0, sem0)

        def ebody(t, carry):
            j0 = 2 * t
            j1 = j0 + 1
            pltpu.async_copy(table_hbm.at[src_v.at[j1]], rows1, sem1)
            pltpu.make_async_copy(dummy, rows0, sem0).wait()
            pltpu.sync_copy(rows0, acc.at[dst_v.at[j0]], add=True)
            if with_counts:
                pltpu.sync_copy(ones_v, cacc.at[dst_v.at[j0]], add=True)

            @pl.when(j1 + 1 < n_chunks)
            def _():
                pltpu.async_copy(table_hbm.at[src_v.at[j1 + 1]], rows0, sem0)

            pltpu.make_async_copy(dummy, rows1, sem1).wait()
            pltpu.sync_copy(rows1, acc.at[dst_v.at[j1]], add=True)
            if with_counts:
                pltpu.sync_copy(ones_v, cacc.at[dst_v.at[j1]], add=True)
            return carry
        lax.fori_loop(0, n_chunks // 2, ebody, 0)

        plsc.subcore_barrier()

        pltpu.sync_copy(acc.at[pl.ds(row0, rows_per_sub)], stage_v)
        pltpu.sync_copy(stage_v,
                        out_hbm.at[pl.ds(c * n_nodes + row0, rows_per_sub)])
        if with_counts:
            pltpu.sync_copy(cacc.at[pl.ds(row0, rows_per_sub)], cstage_v)
            pltpu.sync_copy(cstage_v,
                            cnt_hbm.at[pl.ds(c * n_nodes + row0, rows_per_sub)])

    return segsum


def kernel(x, edge_index, W1l, b1l, W1r, W2l, b2l, W2r):
    ei = jnp.squeeze(edge_index, axis=0) if edge_index.ndim == 3 else edge_index
    src = ei[0].astype(jnp.int32)
    dst = ei[1].astype(jnp.int32)

    n, d_feat = x.shape
    hidden = W1l.shape[0]
    e = src.shape[0]
    n_workers = N_CORE * N_SUB
    per_tile = e // n_workers
    chunk = 100
    n_chunks = per_tile // chunk
    assert per_tile * n_workers == e and n_chunks * chunk == per_tile
    assert n % N_SUB == 0

    src_r = src.reshape(n_workers, n_chunks, chunk)
    dst_r = dst.reshape(n_workers, n_chunks, chunk)

    blk = 1000
    grid = (n // blk,)

    y1, yr = pl.pallas_call(
        _tc1_body,
        grid=grid,
        in_specs=[
            pl.BlockSpec((blk, d_feat), lambda i: (i, 0)),
            pl.BlockSpec((hidden, d_feat), lambda i: (0, 0)),
            pl.BlockSpec((hidden, d_feat), lambda i: (0, 0)),
        ],
        out_specs=[
            pl.BlockSpec((blk, hidden), lambda i: (i, 0)),
            pl.BlockSpec((blk, hidden), lambda i: (i, 0)),
        ],
        out_shape=[
            jax.ShapeDtypeStruct((n, hidden), jnp.float32),
            jax.ShapeDtypeStruct((n, hidden), jnp.float32),
        ],
    )(x, W1l, W1r)

    p1, cnt = _make_sc_segsum(n, hidden, n_chunks, chunk, True)(
        y1, src_r, dst_r)
    p1 = p1.reshape(N_CORE, n, hidden)
    cnt = cnt.reshape(N_CORE, n, 16)

    h, inv = pl.pallas_call(
        _tc2_body,
        grid=grid,
        in_specs=[
            pl.BlockSpec((N_CORE, blk, hidden), lambda i: (0, i, 0)),
            pl.BlockSpec((N_CORE, blk, 16), lambda i: (0, i, 0)),
            pl.BlockSpec((blk, hidden), lambda i: (i, 0)),
            pl.BlockSpec((1, hidden), lambda i: (0, 0)),
        ],
        out_specs=[
            pl.BlockSpec((blk, hidden), lambda i: (i, 0)),
            pl.BlockSpec((blk, 1), lambda i: (i, 0)),
        ],
        out_shape=[
            jax.ShapeDtypeStruct((n, hidden), jnp.float32),
            jax.ShapeDtypeStruct((n, 1), jnp.float32),
        ],
    )(p1, cnt, yr, b1l.reshape(1, hidden))

    (p2,) = _make_sc_segsum(n, hidden, n_chunks, chunk, False)(h, src_r, dst_r)
    p2 = p2.reshape(N_CORE, n, hidden)

    out = pl.pallas_call(
        _tc3_body,
        grid=grid,
        in_specs=[
            pl.BlockSpec((N_CORE, blk, hidden), lambda i: (0, i, 0)),
            pl.BlockSpec((blk, 1), lambda i: (i, 0)),
            pl.BlockSpec((blk, hidden), lambda i: (i, 0)),
            pl.BlockSpec((d_feat, hidden), lambda i: (0, 0)),
            pl.BlockSpec((d_feat, hidden), lambda i: (0, 0)),
            pl.BlockSpec((1, d_feat), lambda i: (0, 0)),
        ],
        out_specs=pl.BlockSpec((blk, d_feat), lambda i: (i, 0)),
        out_shape=jax.ShapeDtypeStruct((n, d_feat), jnp.float32),
    )(p2, inv, h, W2l, W2r, b2l.reshape(1, d_feat))

    return out


# speedup vs baseline: 1.0333x; 1.0333x over previous
import functools

import jax
import jax.numpy as jnp
from jax import lax
from jax.experimental import pallas as pl
from jax.experimental.pallas import tpu as pltpu
from jax.experimental.pallas import tpu_sc as plsc

N_SUB = 16
N_CORE = 2
H = 16


def _tc1_body(x_ref, wl_ref, wr_ref, o1_ref, o2_ref):
    xb = x_ref[...]
    y1 = lax.dot_general(xb, wl_ref[...], (((1,), (1,)), ((), ())),
                         preferred_element_type=jnp.float32)
    yr = lax.dot_general(xb, wr_ref[...], (((1,), (1,)), ((), ())),
                         preferred_element_type=jnp.float32)
    o1_ref[...] = jnp.concatenate([y1, jnp.ones_like(y1)], axis=1)
    o2_ref[...] = yr


def _tc3_body(p1_ref, q_ref, h_ref, wl_ref, wr_ref, b_ref, o_ref):
    cnt = p1_ref[0, :, H:] + p1_ref[1, :, H:]
    inv = 1.0 / jnp.maximum(cnt, 1.0)
    q = q_ref[...]
    m2 = (q[0] + q[1]) * inv
    h = h_ref[...]
    out = lax.dot_general(m2, wl_ref[...], (((1,), (1,)), ((), ())),
                          preferred_element_type=jnp.float32)
    out += lax.dot_general(h, wr_ref[...], (((1,), (1,)), ((), ())),
                           preferred_element_type=jnp.float32)
    o_ref[...] = out + b_ref[...]


def _edge_loop(table_hbm, src_v, dst_v, rows0, rows1, acc, sem0, sem1,
               n_chunks, chunk):
    dummy = table_hbm.at[pl.ds(0, chunk)]
    pltpu.async_copy(table_hbm.at[src_v.at[0]], rows0, sem0)

    def ebody(t, carry):
        j0 = 2 * t
        j1 = j0 + 1
        pltpu.async_copy(table_hbm.at[src_v.at[j1]], rows1, sem1)
        pltpu.make_async_copy(dummy, rows0, sem0).wait()
        pltpu.sync_copy(rows0, acc.at[dst_v.at[j0]], add=True)

        @pl.when(j1 + 1 < n_chunks)
        def _():
            pltpu.async_copy(table_hbm.at[src_v.at[j1 + 1]], rows0, sem0)

        pltpu.make_async_copy(dummy, rows1, sem1).wait()
        pltpu.sync_copy(rows1, acc.at[dst_v.at[j1]], add=True)
        return carry
    lax.fori_loop(0, n_chunks // 2, ebody, 0)


def _make_sc1(n_nodes, n_chunks, chunk):
    d = 2 * H
    rows_per_sub = n_nodes // N_SUB
    mesh = plsc.VectorSubcoreMesh(core_axis_name="c", subcore_axis_name="s")

    @functools.partial(
        pl.kernel, mesh=mesh,
        compiler_params=pltpu.CompilerParams(use_tc_tiling_on_sc=False),
        out_type=jax.ShapeDtypeStruct((N_CORE * n_nodes, d), jnp.float32),
        scratch_types=[
            pltpu.VMEM((n_chunks, chunk), jnp.int32),
            pltpu.VMEM((n_chunks, chunk), jnp.int32),
            pltpu.VMEM((chunk, d), jnp.float32),
            pltpu.VMEM((chunk, d), jnp.float32),
            pltpu.VMEM((rows_per_sub, d), jnp.float32),
            pltpu.VMEM_SHARED((n_nodes, d), jnp.float32),
            pltpu.SemaphoreType.DMA,
            pltpu.SemaphoreType.DMA,
        ],
    )
    def sc1(table_hbm, src_hbm, dst_hbm, out_hbm,
            src_v, dst_v, rows0, rows1, stage_v, acc, sem0, sem1):
        c = lax.axis_index("c")
        s = lax.axis_index("s")
        wid = c * N_SUB + s
        row0 = s * rows_per_sub

        def zbody(i, carry):
            for k in range(d // 16):
                stage_v[i, pl.ds(k * 16, 16)] = jnp.zeros((16,), jnp.float32)
            return carry
        lax.fori_loop(0, rows_per_sub, zbody, 0)
        pltpu.sync_copy(stage_v, acc.at[pl.ds(row0, rows_per_sub)])

        pltpu.sync_copy(src_hbm.at[wid], src_v)
        pltpu.sync_copy(dst_hbm.at[wid], dst_v)

        plsc.subcore_barrier()
        _edge_loop(table_hbm, src_v, dst_v, rows0, rows1, acc, sem0, sem1,
                   n_chunks, chunk)
        plsc.subcore_barrier()

        pltpu.sync_copy(acc.at[pl.ds(row0, rows_per_sub)], stage_v)
        pltpu.sync_copy(stage_v,
                        out_hbm.at[pl.ds(c * n_nodes + row0, rows_per_sub)])

    return sc1


def _make_sc2(n_nodes, n_chunks, chunk):
    d = H
    rows_per_sub = n_nodes // N_SUB
    mesh = plsc.VectorSubcoreMesh(core_axis_name="c", subcore_axis_name="s")

    @functools.partial(
        pl.kernel, mesh=mesh,
        compiler_params=pltpu.CompilerParams(use_tc_tiling_on_sc=False),
        out_type=[
            jax.ShapeDtypeStruct((N_CORE * n_nodes, d), jnp.float32),
            jax.ShapeDtypeStruct((N_CORE * n_nodes, d), jnp.float32),
        ],
        scratch_types=[
            pltpu.VMEM((n_chunks, chunk), jnp.int32),
            pltpu.VMEM((n_chunks, chunk), jnp.int32),
            pltpu.VMEM((chunk, d), jnp.float32),
            pltpu.VMEM((chunk, d), jnp.float32),
            pltpu.VMEM((rows_per_sub, d), jnp.float32),
            pltpu.VMEM((rows_per_sub, 2 * H), jnp.float32),
            pltpu.VMEM((rows_per_sub, 2 * H), jnp.float32),
            pltpu.VMEM((rows_per_sub, H), jnp.float32),
            pltpu.VMEM((H,), jnp.float32),
            pltpu.VMEM_SHARED((n_nodes, d), jnp.float32),
            pltpu.SemaphoreType.DMA,
            pltpu.SemaphoreType.DMA,
        ],
    )
    def sc2(p1_hbm, yr_hbm, b_hbm, src_hbm, dst_hbm, out_hbm, h_hbm,
            src_v, dst_v, rows0, rows1, stage_v, p0_v, p1_v, yr_v, b_v,
            acc, sem0, sem1):
        c = lax.axis_index("c")
        s = lax.axis_index("s")
        wid = c * N_SUB + s
        row0 = s * rows_per_sub

        pltpu.sync_copy(p1_hbm.at[pl.ds(row0, rows_per_sub)], p0_v)
        pltpu.sync_copy(p1_hbm.at[pl.ds(n_nodes + row0, rows_per_sub)], p1_v)
        pltpu.sync_copy(yr_hbm.at[pl.ds(row0, rows_per_sub)], yr_v)
        pltpu.sync_copy(b_hbm, b_v)
        bvec = b_v[...]

        def hbody(i, carry):
            sm = p0_v[i, pl.ds(0, 16)] + p1_v[i, pl.ds(0, 16)]
            cnt = p0_v[i, pl.ds(16, 16)] + p1_v[i, pl.ds(16, 16)]
            inv = 1.0 / jnp.maximum(cnt, 1.0)
            h = jnp.maximum(sm * inv + bvec + yr_v[i, pl.ds(0, 16)], 0.0)
            stage_v[i, pl.ds(0, 16)] = h
            return carry
        lax.fori_loop(0, rows_per_sub, hbody, 0)
        pltpu.sync_copy(stage_v,
                        h_hbm.at[pl.ds(c * n_nodes + row0, rows_per_sub)])

        def zbody(i, carry):
            stage_v[i, pl.ds(0, 16)] = jnp.zeros((16,), jnp.float32)
            return carry
        lax.fori_loop(0, rows_per_sub, zbody, 0)
        pltpu.sync_copy(stage_v, acc.at[pl.ds(row0, rows_per_sub)])

        pltpu.sync_copy(src_hbm.at[wid], src_v)
        pltpu.sync_copy(dst_hbm.at[wid], dst_v)

        plsc.subcore_barrier()
        _edge_loop(h_hbm, src_v, dst_v, rows0, rows1, acc, sem0, sem1,
                   n_chunks, chunk)
        plsc.subcore_barrier()

        pltpu.sync_copy(acc.at[pl.ds(row0, rows_per_sub)], stage_v)
        pltpu.sync_copy(stage_v,
                        out_hbm.at[pl.ds(c * n_nodes + row0, rows_per_sub)])

    return sc2


def kernel(x, edge_index, W1l, b1l, W1r, W2l, b2l, W2r):
    ei = jnp.squeeze(edge_index, axis=0) if edge_index.ndim == 3 else edge_index
    src = ei[0].astype(jnp.int32)
    dst = ei[1].astype(jnp.int32)

    n, d_feat = x.shape
    hidden = W1l.shape[0]
    e = src.shape[0]
    n_workers = N_CORE * N_SUB
    per_tile = e // n_workers
    chunk = 100
    n_chunks = per_tile // chunk
    assert per_tile * n_workers == e and n_chunks * chunk == per_tile
    assert n % N_SUB == 0 and n_chunks % 2 == 0 and hidden == H

    src_r = src.reshape(n_workers, n_chunks, chunk)
    dst_r = dst.reshape(n_workers, n_chunks, chunk)
    core_off = (jnp.arange(n_workers, dtype=jnp.int32) // N_SUB) * n
    src2_r = src_r + core_off.reshape(n_workers, 1, 1)

    blk = 1000
    grid = (n // blk,)

    y1aug, yr = pl.pallas_call(
        _tc1_body,
        grid=grid,
        in_specs=[
            pl.BlockSpec((blk, d_feat), lambda i: (i, 0)),
            pl.BlockSpec((hidden, d_feat), lambda i: (0, 0)),
            pl.BlockSpec((hidden, d_feat), lambda i: (0, 0)),
        ],
        out_specs=[
            pl.BlockSpec((blk, 2 * hidden), lambda i: (i, 0)),
            pl.BlockSpec((blk, hidden), lambda i: (i, 0)),
        ],
        out_shape=[
            jax.ShapeDtypeStruct((n, 2 * hidden), jnp.float32),
            jax.ShapeDtypeStruct((n, hidden), jnp.float32),
        ],
    )(x, W1l, W1r)

    p1 = _make_sc1(n, n_chunks, chunk)(y1aug, src_r, dst_r)

    p2, h2 = _make_sc2(n, n_chunks, chunk)(p1, yr, b1l, src2_r, dst_r)
    h = h2[:n]

    p1r = p1.reshape(N_CORE, n, 2 * hidden)
    p2r = p2.reshape(N_CORE, n, hidden)
    out = pl.pallas_call(
        _tc3_body,
        grid=grid,
        in_specs=[
            pl.BlockSpec((N_CORE, blk, 2 * hidden), lambda i: (0, i, 0)),
            pl.BlockSpec((N_CORE, blk, hidden), lambda i: (0, i, 0)),
            pl.BlockSpec((blk, hidden), lambda i: (i, 0)),
            pl.BlockSpec((d_feat, hidden), lambda i: (0, 0)),
            pl.BlockSpec((d_feat, hidden), lambda i: (0, 0)),
            pl.BlockSpec((1, d_feat), lambda i: (0, 0)),
        ],
        out_specs=pl.BlockSpec((blk, d_feat), lambda i: (i, 0)),
        out_shape=jax.ShapeDtypeStruct((n, d_feat), jnp.float32),
    )(p1r, p2r, h, W2l, W2r, b2l.reshape(1, d_feat))

    return out


# speedup vs baseline: 1.3178x; 1.2754x over previous
import functools

import jax
import jax.numpy as jnp
from jax import lax
from jax.experimental import pallas as pl
from jax.experimental.pallas import tpu as pltpu
from jax.experimental.pallas import tpu_sc as plsc

N_SUB = 16
N_CORE = 2
H = 16


def _tc1_body(x_ref, wl_ref, wr_ref, o1_ref, o2_ref):
    xb = x_ref[...]
    y1 = lax.dot_general(xb, wl_ref[...], (((1,), (1,)), ((), ())),
                         preferred_element_type=jnp.float32)
    yr = lax.dot_general(xb, wr_ref[...], (((1,), (1,)), ((), ())),
                         preferred_element_type=jnp.float32)
    o1_ref[...] = jnp.concatenate([y1, jnp.ones_like(y1)], axis=1)
    o2_ref[...] = yr


def _tc3_body(p1_ref, q_ref, h_ref, wl_ref, wr_ref, b_ref, o_ref):
    cnt = p1_ref[0, :, H:] + p1_ref[1, :, H:]
    inv = 1.0 / jnp.maximum(cnt, 1.0)
    q = q_ref[...]
    m2 = (q[0] + q[1]) * inv
    h = h_ref[...]
    out = lax.dot_general(m2, wl_ref[...], (((1,), (1,)), ((), ())),
                          preferred_element_type=jnp.float32)
    out += lax.dot_general(h, wr_ref[...], (((1,), (1,)), ((), ())),
                           preferred_element_type=jnp.float32)
    o_ref[...] = out + b_ref[...]


NBUF = 4


def _edge_loop(table_hbm, src_v, dst_v, rows, acc, sems, n_chunks, chunk):
    dummy = table_hbm.at[pl.ds(0, chunk)]
    for b in range(NBUF):
        pltpu.async_copy(table_hbm.at[src_v.at[b]], rows[b], sems[b])

    def ebody(t, carry):
        for b in range(NBUF):
            j = NBUF * t + b
            pltpu.make_async_copy(dummy, rows[b], sems[b]).wait()
            pltpu.sync_copy(rows[b], acc.at[dst_v.at[j]], add=True)

            @pl.when(j + NBUF < n_chunks)
            def _():
                pltpu.async_copy(table_hbm.at[src_v.at[j + NBUF]],
                                 rows[b], sems[b])
        return carry
    lax.fori_loop(0, n_chunks // NBUF, ebody, 0)


def _make_sc1(n_nodes, n_chunks, chunk):
    d = 2 * H
    rows_per_sub = n_nodes // N_SUB
    mesh = plsc.VectorSubcoreMesh(core_axis_name="c", subcore_axis_name="s")

    @functools.partial(
        pl.kernel, mesh=mesh,
        compiler_params=pltpu.CompilerParams(use_tc_tiling_on_sc=False),
        out_type=jax.ShapeDtypeStruct((N_CORE * n_nodes, d), jnp.float32),
        scratch_types=[
            pltpu.VMEM((n_chunks, chunk), jnp.int32),
            pltpu.VMEM((n_chunks, chunk), jnp.int32),
            pltpu.VMEM((chunk, d), jnp.float32),
            pltpu.VMEM((chunk, d), jnp.float32),
            pltpu.VMEM((chunk, d), jnp.float32),
            pltpu.VMEM((chunk, d), jnp.float32),
            pltpu.VMEM((rows_per_sub, d), jnp.float32),
            pltpu.VMEM_SHARED((n_nodes, d), jnp.float32),
            pltpu.SemaphoreType.DMA,
            pltpu.SemaphoreType.DMA,
            pltpu.SemaphoreType.DMA,
            pltpu.SemaphoreType.DMA,
        ],
    )
    def sc1(table_hbm, src_hbm, dst_hbm, out_hbm,
            src_v, dst_v, r0, r1, r2, r3, stage_v, acc, s0, s1, s2, s3):
        c = lax.axis_index("c")
        s = lax.axis_index("s")
        wid = c * N_SUB + s
        row0 = s * rows_per_sub

        def zbody(i, carry):
            for k in range(d // 16):
                stage_v[i, pl.ds(k * 16, 16)] = jnp.zeros((16,), jnp.float32)
            return carry
        lax.fori_loop(0, rows_per_sub, zbody, 0)
        pltpu.sync_copy(stage_v, acc.at[pl.ds(row0, rows_per_sub)])

        pltpu.sync_copy(src_hbm.at[wid], src_v)
        pltpu.sync_copy(dst_hbm.at[wid], dst_v)

        plsc.subcore_barrier()
        _edge_loop(table_hbm, src_v, dst_v, [r0, r1, r2, r3], acc,
                   [s0, s1, s2, s3], n_chunks, chunk)
        plsc.subcore_barrier()

        pltpu.sync_copy(acc.at[pl.ds(row0, rows_per_sub)], stage_v)
        pltpu.sync_copy(stage_v,
                        out_hbm.at[pl.ds(c * n_nodes + row0, rows_per_sub)])

    return sc1


def _make_sc2(n_nodes, n_chunks, chunk):
    d = H
    rows_per_sub = n_nodes // N_SUB
    mesh = plsc.VectorSubcoreMesh(core_axis_name="c", subcore_axis_name="s")

    @functools.partial(
        pl.kernel, mesh=mesh,
        compiler_params=pltpu.CompilerParams(use_tc_tiling_on_sc=False),
        out_type=[
            jax.ShapeDtypeStruct((N_CORE * n_nodes, d), jnp.float32),
            jax.ShapeDtypeStruct((N_CORE * n_nodes, d), jnp.float32),
        ],
        scratch_types=[
            pltpu.VMEM((n_chunks, chunk), jnp.int32),
            pltpu.VMEM((n_chunks, chunk), jnp.int32),
            pltpu.VMEM((chunk, d), jnp.float32),
            pltpu.VMEM((chunk, d), jnp.float32),
            pltpu.VMEM((chunk, d), jnp.float32),
            pltpu.VMEM((chunk, d), jnp.float32),
            pltpu.VMEM((rows_per_sub, d), jnp.float32),
            pltpu.VMEM((rows_per_sub, 2 * H), jnp.float32),
            pltpu.VMEM((rows_per_sub, 2 * H), jnp.float32),
            pltpu.VMEM((rows_per_sub, H), jnp.float32),
            pltpu.VMEM((H,), jnp.float32),
            pltpu.VMEM_SHARED((n_nodes, d), jnp.float32),
            pltpu.SemaphoreType.DMA,
            pltpu.SemaphoreType.DMA,
            pltpu.SemaphoreType.DMA,
            pltpu.SemaphoreType.DMA,
        ],
    )
    def sc2(p1_hbm, yr_hbm, b_hbm, src_hbm, dst_hbm, out_hbm, h_hbm,
            src_v, dst_v, r0, r1, r2, r3, stage_v, p0_v, p1_v, yr_v, b_v,
            acc, s0, s1, s2, s3):
        c = lax.axis_index("c")
        s = lax.axis_index("s")
        wid = c * N_SUB + s
        row0 = s * rows_per_sub

        pltpu.sync_copy(p1_hbm.at[pl.ds(row0, rows_per_sub)], p0_v)
        pltpu.sync_copy(p1_hbm.at[pl.ds(n_nodes + row0, rows_per_sub)], p1_v)
        pltpu.sync_copy(yr_hbm.at[pl.ds(row0, rows_per_sub)], yr_v)
        pltpu.sync_copy(b_hbm, b_v)
        bvec = b_v[...]

        def hbody(i, carry):
            sm = p0_v[i, pl.ds(0, 16)] + p1_v[i, pl.ds(0, 16)]
            cnt = p0_v[i, pl.ds(16, 16)] + p1_v[i, pl.ds(16, 16)]
            inv = 1.0 / jnp.maximum(cnt, 1.0)
            h = jnp.maximum(sm * inv + bvec + yr_v[i, pl.ds(0, 16)], 0.0)
            stage_v[i, pl.ds(0, 16)] = h
            return carry
        lax.fori_loop(0, rows_per_sub, hbody, 0)
        pltpu.sync_copy(stage_v,
                        h_hbm.at[pl.ds(c * n_nodes + row0, rows_per_sub)])

        def zbody(i, carry):
            stage_v[i, pl.ds(0, 16)] = jnp.zeros((16,), jnp.float32)
            return carry
        lax.fori_loop(0, rows_per_sub, zbody, 0)
        pltpu.sync_copy(stage_v, acc.at[pl.ds(row0, rows_per_sub)])

        pltpu.sync_copy(src_hbm.at[wid], src_v)
        pltpu.sync_copy(dst_hbm.at[wid], dst_v)

        plsc.subcore_barrier()
        _edge_loop(h_hbm, src_v, dst_v, [r0, r1, r2, r3], acc,
                   [s0, s1, s2, s3], n_chunks, chunk)
        plsc.subcore_barrier()

        pltpu.sync_copy(acc.at[pl.ds(row0, rows_per_sub)], stage_v)
        pltpu.sync_copy(stage_v,
                        out_hbm.at[pl.ds(c * n_nodes + row0, rows_per_sub)])

    return sc2


def kernel(x, edge_index, W1l, b1l, W1r, W2l, b2l, W2r):
    ei = jnp.squeeze(edge_index, axis=0) if edge_index.ndim == 3 else edge_index
    src = ei[0].astype(jnp.int32)
    dst = ei[1].astype(jnp.int32)

    n, d_feat = x.shape
    hidden = W1l.shape[0]
    e = src.shape[0]
    n_workers = N_CORE * N_SUB
    per_tile = e // n_workers
    chunk = 125
    n_chunks = per_tile // chunk
    assert per_tile * n_workers == e and n_chunks * chunk == per_tile
    assert n % N_SUB == 0 and n_chunks % 4 == 0 and hidden == H

    src_r = src.reshape(n_workers, n_chunks, chunk)
    dst_r = dst.reshape(n_workers, n_chunks, chunk)
    core_off = (jnp.arange(n_workers, dtype=jnp.int32) // N_SUB) * n
    src2_r = src_r + core_off.reshape(n_workers, 1, 1)

    blk = 1000
    grid = (n // blk,)

    y1aug, yr = pl.pallas_call(
        _tc1_body,
        grid=grid,
        in_specs=[
            pl.BlockSpec((blk, d_feat), lambda i: (i, 0)),
            pl.BlockSpec((hidden, d_feat), lambda i: (0, 0)),
            pl.BlockSpec((hidden, d_feat), lambda i: (0, 0)),
        ],
        out_specs=[
            pl.BlockSpec((blk, 2 * hidden), lambda i: (i, 0)),
            pl.BlockSpec((blk, hidden), lambda i: (i, 0)),
        ],
        out_shape=[
            jax.ShapeDtypeStruct((n, 2 * hidden), jnp.float32),
            jax.ShapeDtypeStruct((n, hidden), jnp.float32),
        ],
    )(x, W1l, W1r)

    p1 = _make_sc1(n, n_chunks, chunk)(y1aug, src_r, dst_r)

    p2, h2 = _make_sc2(n, n_chunks, chunk)(p1, yr, b1l, src2_r, dst_r)
    h = h2[:n]

    p1r = p1.reshape(N_CORE, n, 2 * hidden)
    p2r = p2.reshape(N_CORE, n, hidden)
    out = pl.pallas_call(
        _tc3_body,
        grid=grid,
        in_specs=[
            pl.BlockSpec((N_CORE, blk, 2 * hidden), lambda i: (0, i, 0)),
            pl.BlockSpec((N_CORE, blk, hidden), lambda i: (0, i, 0)),
            pl.BlockSpec((blk, hidden), lambda i: (i, 0)),
            pl.BlockSpec((d_feat, hidden), lambda i: (0, 0)),
            pl.BlockSpec((d_feat, hidden), lambda i: (0, 0)),
            pl.BlockSpec((1, d_feat), lambda i: (0, 0)),
        ],
        out_specs=pl.BlockSpec((blk, d_feat), lambda i: (i, 0)),
        out_shape=jax.ShapeDtypeStruct((n, d_feat), jnp.float32),
    )(p1r, p2r, h, W2l, W2r, b2l.reshape(1, d_feat))

    return out
